# baseline (speedup 1.0000x reference)
import jax
import jax.numpy as jnp
from jax.experimental import pallas as pl
from jax.experimental.pallas import tpu as pltpu

WORDLEN = 100000
EMB = 64
CTX = 20
HID = 128
BK = 2048
NJ = (WORDLEN + BK - 1) // BK
PAD = NJ * BK
NSTEPS = CTX + NJ


def _fused(x_ref, table_blk, w1_blk, b1_blk, w2_blk, b2_blk, out_ref,
           h_ref, m_ref, s_ref):
    i = pl.program_id(0)

    @pl.when(i == 0)
    def _init():
        h_ref[...] = b1_blk[...]
        m_ref[0] = -jnp.inf
        s_ref[0] = 0.0

    @pl.when(i < CTX)
    def _accum_h():
        h_ref[...] += jnp.dot(table_blk[0], w1_blk[0],
                              preferred_element_type=jnp.float32)

    @pl.when(i == CTX - 1)
    def _relu():
        h_ref[...] = jnp.maximum(h_ref[...], 0.0)

    @pl.when(i >= CTX)
    def _vocab_block():
        j = i - CTX
        logits = jnp.dot(h_ref[...], w2_blk[...],
                         preferred_element_type=jnp.float32) + b2_blk[...]
        col = jax.lax.broadcasted_iota(jnp.int32, (1, BK), 1) + j * BK
        logits = jnp.where(col < WORDLEN, logits, -jnp.inf)
        out_ref[:, pl.ds(j * BK, BK)] = logits
        m_old = m_ref[0]
        s_old = s_ref[0]
        m_new = jnp.maximum(m_old, jnp.max(logits))
        s_new = s_old * jnp.exp(m_old - m_new) + jnp.sum(jnp.exp(logits - m_new))
        m_ref[0] = m_new
        s_ref[0] = s_new

        @pl.when(j == NJ - 1)
        def _finalize():
            lse = m_ref[0] + jnp.log(s_ref[0])
            out_ref[...] = out_ref[...] - lse


def kernel(x, table, W1, b1, W2, b2):
    table3 = table.reshape(WORDLEN, 1, EMB)
    w1r = W1.reshape(CTX, EMB, HID)
    b1r = b1.reshape(1, HID)
    b2r = b2.reshape(1, WORDLEN)

    grid_spec = pltpu.PrefetchScalarGridSpec(
        num_scalar_prefetch=1,
        grid=(NSTEPS,),
        in_specs=[
            pl.BlockSpec((1, 1, EMB),
                         lambda i, xr: (xr[jnp.minimum(i, CTX - 1)], 0, 0)),
            pl.BlockSpec((1, EMB, HID),
                         lambda i, xr: (jnp.minimum(i, CTX - 1), 0, 0)),
            pl.BlockSpec((1, HID), lambda i, xr: (0, 0)),
            pl.BlockSpec((HID, BK), lambda i, xr: (0, jnp.maximum(i - CTX, 0))),
            pl.BlockSpec((1, BK), lambda i, xr: (0, jnp.maximum(i - CTX, 0))),
        ],
        out_specs=pl.BlockSpec((1, PAD), lambda i, xr: (0, 0)),
        scratch_shapes=[
            pltpu.VMEM((1, HID), jnp.float32),
            pltpu.SMEM((1,), jnp.float32),
            pltpu.SMEM((1,), jnp.float32),
        ],
    )

    out = pl.pallas_call(
        _fused,
        grid_spec=grid_spec,
        out_shape=jax.ShapeDtypeStruct((1, PAD), jnp.float32),
    )(x, table3, w1r, b1r, W2, b2r)
    return out[:, :WORDLEN]


# speedup vs baseline: 1.2301x; 1.2301x over previous
import jax
import jax.numpy as jnp
from jax.experimental import pallas as pl
from jax.experimental.pallas import tpu as pltpu

WORDLEN = 100000
EMB = 64
CTX = 20
HID = 128
BK = 8192
NJ = (WORDLEN + BK - 1) // BK
PAD = NJ * BK
NEG = -jnp.inf


def _fused(x_ref, table_hbm, w1_ref, b1_ref, w2_blk, b2_blk, out_ref,
           emb_ref, h_ref, m_ref, s_ref, sem):
    j = pl.program_id(0)

    @pl.when(j == 0)
    def _gather_and_h():
        copies = []
        for i in range(CTX):
            c = pltpu.make_async_copy(
                table_hbm.at[pl.ds(x_ref[i], 1)],
                emb_ref.at[pl.ds(i, 1)],
                sem,
            )
            c.start()
            copies.append(c)
        for c in copies:
            c.wait()
        acc = b1_ref[...]
        for i in range(CTX):
            acc = acc + jnp.dot(emb_ref[i:i + 1, :],
                                w1_ref[i * EMB:(i + 1) * EMB, :],
                                preferred_element_type=jnp.float32)
        h_ref[...] = jnp.maximum(acc, 0.0)
        m_ref[...] = jnp.full((1, BK), NEG, jnp.float32)
        s_ref[...] = jnp.zeros((1, BK), jnp.float32)

    logits = jnp.dot(h_ref[...], w2_blk[...],
                     preferred_element_type=jnp.float32) + b2_blk[...]

    @pl.when(j == NJ - 1)
    def _mask_tail():
        col = jax.lax.broadcasted_iota(jnp.int32, (1, BK), 1) + j * BK
        out_ref[:, pl.ds(j * BK, BK)] = jnp.where(col < WORDLEN, logits, NEG)

    @pl.when(j < NJ - 1)
    def _store():
        out_ref[:, pl.ds(j * BK, BK)] = logits

    ln = out_ref[:, pl.ds(j * BK, BK)]
    m_old = m_ref[...]
    m_new = jnp.maximum(m_old, ln)
    s_ref[...] = s_ref[...] * jnp.exp(m_old - m_new) + jnp.exp(ln - m_new)
    m_ref[...] = m_new

    @pl.when(j == NJ - 1)
    def _finalize():
        m = jnp.max(m_ref[...])
        s = jnp.sum(s_ref[...] * jnp.exp(m_ref[...] - m))
        out_ref[...] = out_ref[...] - (m + jnp.log(s))


def kernel(x, table, W1, b1, W2, b2):
    b1r = b1.reshape(1, HID)
    b2r = b2.reshape(1, WORDLEN)

    grid_spec = pltpu.PrefetchScalarGridSpec(
        num_scalar_prefetch=1,
        grid=(NJ,),
        in_specs=[
            pl.BlockSpec(memory_space=pl.ANY),
            pl.BlockSpec((HID * 10, HID), lambda j, xr: (0, 0)),
            pl.BlockSpec((1, HID), lambda j, xr: (0, 0)),
            pl.BlockSpec((HID, BK), lambda j, xr: (0, j)),
            pl.BlockSpec((1, BK), lambda j, xr: (0, j)),
        ],
        out_specs=pl.BlockSpec((1, PAD), lambda j, xr: (0, 0)),
        scratch_shapes=[
            pltpu.VMEM((CTX, EMB), jnp.float32),
            pltpu.VMEM((1, HID), jnp.float32),
            pltpu.VMEM((1, BK), jnp.float32),
            pltpu.VMEM((1, BK), jnp.float32),
            pltpu.SemaphoreType.DMA,
        ],
    )

    out = pl.pallas_call(
        _fused,
        grid_spec=grid_spec,
        out_shape=jax.ShapeDtypeStruct((1, PAD), jnp.float32),
    )(x, table, W1, b1r, W2, b2r)
    return out[:, :WORDLEN]
